# baseline (speedup 1.0000x reference)
import functools

import jax
import jax.numpy as jnp
from jax import lax
from jax.experimental import pallas as pl
from jax.experimental.pallas import tpu as pltpu
from jax.experimental.pallas import tpu_sc as plsc

_BLK_N = 512
_NW = 32
_CH = 128


def _assign_kernel(x_ref, cb_ref, idx_ref, loss_ref, hcn_ref):
    i = pl.program_id(0)

    @pl.when(i == 0)
    def _():
        cb0 = cb_ref[:]
        hcn_ref[0, :] = 0.5 * jnp.sum(cb0 * cb0, axis=1)

    x = x_ref[:]
    s = lax.dot_general(
        x, cb_ref[:], (((1,), (1,)), ((), ())),
        preferred_element_type=jnp.float32,
    )
    s = s - hcn_ref[:]
    m = jnp.max(s, axis=-1)
    idx_ref[0, 0, :] = jnp.argmax(s, axis=-1).astype(jnp.int32)
    part = jnp.sum(x * x) - 2.0 * jnp.sum(m)

    @pl.when(i == 0)
    def _():
        loss_ref[0, 0] = part

    @pl.when(i > 0)
    def _():
        loss_ref[0, 0] += part


def _make_gather(n: int, d: int):
    tok_per_w = n // _NW
    nch = tok_per_w // _CH
    mesh = plsc.VectorSubcoreMesh(core_axis_name="c", subcore_axis_name="s")

    @functools.partial(
        pl.kernel,
        mesh=mesh,
        out_type=jax.ShapeDtypeStruct((n, d), jnp.float32),
        scratch_types=[
            pltpu.VMEM((nch, _CH), jnp.int32),
            pltpu.VMEM((2, _CH, d), jnp.float32),
            pltpu.SemaphoreType.DMA,
        ],
    )
    def gather_k(cb_hbm, idx_hbm, out_hbm, idx_v, rows_v, sem):
        wid = lax.axis_index("s") * 2 + lax.axis_index("c")
        rbase = wid * nch
        tbase = wid * tok_per_w
        pltpu.sync_copy(idx_hbm.at[pl.ds(rbase, nch)], idx_v)
        handles = [None, None]
        handles[0] = pltpu.async_copy(
            cb_hbm.at[idx_v.at[0]], rows_v.at[0], sem)
        for j in range(nch):
            handles[j % 2].wait()
            if j + 1 < nch:
                handles[(j + 1) % 2] = pltpu.async_copy(
                    cb_hbm.at[idx_v.at[j + 1]], rows_v.at[(j + 1) % 2], sem)
            pltpu.sync_copy(rows_v.at[j % 2],
                            out_hbm.at[pl.ds(tbase + j * _CH, _CH)])

    return gather_k


def kernel(motion_input, codebook):
    b, t, d = motion_input.shape
    k = codebook.shape[0]
    n = b * t
    nb = n // _BLK_N
    flat = motion_input.reshape(n, d)

    idx, loss_sum = pl.pallas_call(
        _assign_kernel,
        grid=(nb,),
        in_specs=[
            pl.BlockSpec((_BLK_N, d), lambda i: (i, 0)),
            pl.BlockSpec((k, d), lambda i: (0, 0)),
        ],
        out_specs=[
            pl.BlockSpec((1, 1, _BLK_N), lambda i: (i, 0, 0)),
            pl.BlockSpec(memory_space=pltpu.SMEM),
        ],
        out_shape=[
            jax.ShapeDtypeStruct((nb, 1, _BLK_N), jnp.int32),
            jax.ShapeDtypeStruct((1, 1), jnp.float32),
        ],
        scratch_shapes=[pltpu.VMEM((1, k), jnp.float32)],
    )(flat, codebook)

    idx_flat = idx.reshape(n)
    q = _make_gather(n, d)(codebook, idx_flat.reshape(n // _CH, _CH))

    quantize = q.reshape(b, t, d)
    embed_ind = idx_flat.reshape(b, t)
    loss = loss_sum[0, 0] / jnp.float32(n * d)
    return (quantize, embed_ind, loss)


# speedup vs baseline: 1.2059x; 1.2059x over previous
import functools

import jax
import jax.numpy as jnp
from jax import lax
from jax.experimental import pallas as pl
from jax.experimental.pallas import tpu as pltpu
from jax.experimental.pallas import tpu_sc as plsc

_BLK_N = 512
_NW = 32
_CH = 128


def _assign_kernel(x_ref, cb_ref, idx_ref, loss_ref, hcn_ref):
    i = pl.program_id(0)

    @pl.when(i == 0)
    def _():
        cb0 = cb_ref[:]
        hcn_ref[:] = 0.5 * jnp.sum(cb0 * cb0, axis=1, keepdims=True)

    x = x_ref[:]
    s = lax.dot_general(
        cb_ref[:], x, (((1,), (1,)), ((), ())),
        preferred_element_type=jnp.float32,
    )
    s = s - hcn_ref[:]
    m = jnp.max(s, axis=0)
    idx_ref[0, 0, :] = jnp.argmax(s, axis=0).astype(jnp.int32)
    part = jnp.sum(x * x) - 2.0 * jnp.sum(m)

    @pl.when(i == 0)
    def _():
        loss_ref[0, 0] = part

    @pl.when(i > 0)
    def _():
        loss_ref[0, 0] += part


def _make_gather(n: int, d: int):
    tok_per_w = n // _NW
    nch = tok_per_w // _CH
    mesh = plsc.VectorSubcoreMesh(core_axis_name="c", subcore_axis_name="s")

    @functools.partial(
        pl.kernel,
        mesh=mesh,
        out_type=jax.ShapeDtypeStruct((n, d), jnp.float32),
        scratch_types=[
            pltpu.VMEM((nch, _CH), jnp.int32),
            pltpu.VMEM((2, _CH, d), jnp.float32),
            pltpu.SemaphoreType.DMA,
        ],
    )
    def gather_k(cb_hbm, idx_hbm, out_hbm, idx_v, rows_v, sem):
        wid = lax.axis_index("s") * 2 + lax.axis_index("c")
        rbase = wid * nch
        tbase = wid * tok_per_w
        pltpu.sync_copy(idx_hbm.at[pl.ds(rbase, nch)], idx_v)
        handles = [None, None]
        handles[0] = pltpu.async_copy(
            cb_hbm.at[idx_v.at[0]], rows_v.at[0], sem)
        for j in range(nch):
            handles[j % 2].wait()
            if j + 1 < nch:
                handles[(j + 1) % 2] = pltpu.async_copy(
                    cb_hbm.at[idx_v.at[j + 1]], rows_v.at[(j + 1) % 2], sem)
            pltpu.sync_copy(rows_v.at[j % 2],
                            out_hbm.at[pl.ds(tbase + j * _CH, _CH)])

    return gather_k


def kernel(motion_input, codebook):
    b, t, d = motion_input.shape
    k = codebook.shape[0]
    n = b * t
    nb = n // _BLK_N
    flat = motion_input.reshape(n, d)

    idx, loss_sum = pl.pallas_call(
        _assign_kernel,
        grid=(nb,),
        in_specs=[
            pl.BlockSpec((_BLK_N, d), lambda i: (i, 0)),
            pl.BlockSpec((k, d), lambda i: (0, 0)),
        ],
        out_specs=[
            pl.BlockSpec((1, 1, _BLK_N), lambda i: (i, 0, 0)),
            pl.BlockSpec(memory_space=pltpu.SMEM),
        ],
        out_shape=[
            jax.ShapeDtypeStruct((nb, 1, _BLK_N), jnp.int32),
            jax.ShapeDtypeStruct((1, 1), jnp.float32),
        ],
        scratch_shapes=[pltpu.VMEM((k, 1), jnp.float32)],
    )(flat, codebook)

    idx_flat = idx.reshape(n)
    q = _make_gather(n, d)(codebook, idx_flat.reshape(n // _CH, _CH))

    quantize = q.reshape(b, t, d)
    embed_ind = idx_flat.reshape(b, t)
    loss = loss_sum[0, 0] / jnp.float32(n * d)
    return (quantize, embed_ind, loss)
